# baseline (speedup 1.0000x reference)
import functools

import jax
import jax.numpy as jnp
from jax import lax
from jax.experimental import pallas as pl
from jax.experimental.pallas import tpu as pltpu
from jax.experimental.pallas import tpu_sc as plsc

N = 100000
E = 1600000
NBKT = 64
WBKT = 1568
NPROD = 32
CAP = 1024
TOT = NBKT * NPROD * CAP
EPP = E // NPROD
NPAD = NBKT * WBKT
NS = 100096
PT = 6256

_mesh = plsc.VectorSubcoreMesh(core_axis_name="c", subcore_axis_name="s")
_sc_params = pltpu.CompilerParams(needs_layout_passes=False,
                                  use_tc_tiling_on_sc=False)


def _i16(v):
    return jnp.full((16,), v, jnp.int32)


def _iota16():
    return lax.iota(jnp.int32, 16)


def _t1_body(pos_ref, x16_ref, msg_ref):
    p = pos_ref[...]
    x16 = jnp.concatenate([p, jnp.zeros((p.shape[0], 13), p.dtype)], axis=1)
    x16_ref[...] = x16
    msg_ref[...] = jnp.maximum(x16, 0.0) + 1e-7


def _t1(pos):
    return pl.pallas_call(
        _t1_body, grid=(50,),
        in_specs=[pl.BlockSpec((2000, 3), lambda i: (i, 0))],
        out_specs=[pl.BlockSpec((2000, 16), lambda i: (i, 0)),
                   pl.BlockSpec((2000, 16), lambda i: (i, 0))],
        out_shape=[jax.ShapeDtypeStruct((N, 16), jnp.float32),
                   jax.ShapeDtypeStruct((N, 16), jnp.float32)],
    )(pos)


def _t2_body(num_ref, x_ref, wa_ref, ba_ref, out_ref, h_ref):
    out = num_ref[...] + x_ref[...]
    out_ref[...] = out
    h_ref[...] = jnp.dot(out, wa_ref[...]) + ba_ref[...]


def _t2(num, x, wa, ba):
    H = wa.shape[1]
    return pl.pallas_call(
        _t2_body, grid=(50,),
        in_specs=[pl.BlockSpec((2000, 16), lambda i: (i, 0)),
                  pl.BlockSpec((2000, 16), lambda i: (i, 0)),
                  pl.BlockSpec((16, H), lambda i: (0, 0)),
                  pl.BlockSpec((1, H), lambda i: (0, 0))],
        out_specs=[pl.BlockSpec((2000, 16), lambda i: (i, 0)),
                   pl.BlockSpec((2000, H), lambda i: (i, 0))],
        out_shape=[jax.ShapeDtypeStruct((N, 16), jnp.float32),
                   jax.ShapeDtypeStruct((N, H), jnp.float32)],
    )(num, x, wa, ba)


def _t3_body_l1(h_ref, mu_ref, sig_ref, g_ref, be_ref, wb_ref, bb_ref,
                x_ref, msg_ref):
    h = h_ref[...]
    hn = (h - mu_ref[...]) / sig_ref[...] * g_ref[...] + be_ref[...]
    hr = jnp.maximum(hn, 0.0)
    xn = jnp.dot(hr, wb_ref[...]) + bb_ref[...]
    xl = jnp.where(xn > 0, xn, 0.01 * xn)
    x_ref[...] = xl
    msg_ref[...] = jnp.maximum(xl, 0.0) + 1e-7


def _t3_l1(h, mu, sig, g, be, wb, bb):
    H = h.shape[1]
    return pl.pallas_call(
        _t3_body_l1, grid=(50,),
        in_specs=[pl.BlockSpec((2000, H), lambda i: (i, 0)),
                  pl.BlockSpec((1, H), lambda i: (0, 0)),
                  pl.BlockSpec((1, H), lambda i: (0, 0)),
                  pl.BlockSpec((1, H), lambda i: (0, 0)),
                  pl.BlockSpec((1, H), lambda i: (0, 0)),
                  pl.BlockSpec((H, 16), lambda i: (0, 0)),
                  pl.BlockSpec((1, 16), lambda i: (0, 0))],
        out_specs=[pl.BlockSpec((2000, 16), lambda i: (i, 0)),
                   pl.BlockSpec((2000, 16), lambda i: (i, 0))],
        out_shape=[jax.ShapeDtypeStruct((N, 16), jnp.float32),
                   jax.ShapeDtypeStruct((N, 16), jnp.float32)],
    )(h, mu, sig, g, be, wb, bb)


def _t3_body_l2(h_ref, mu_ref, sig_ref, g_ref, be_ref, wb_ref, bb_ref,
                wp_ref, nrm_ref, x_ref, sc_ref):
    h = h_ref[...]
    hn = (h - mu_ref[...]) / sig_ref[...] * g_ref[...] + be_ref[...]
    hr = jnp.maximum(hn, 0.0)
    xn = jnp.dot(hr, wb_ref[...]) + bb_ref[...]
    xl = jnp.where(xn > 0, xn, 0.01 * xn)
    x_ref[...] = xl
    d = jnp.dot(xl, wp_ref[...])
    sc_ref[...] = jnp.tanh(d / nrm_ref[0, 0])


def _t3_l2(h, mu, sig, g, be, wb, bb, wp, nrm):
    H = h.shape[1]
    return pl.pallas_call(
        _t3_body_l2, grid=(50,),
        in_specs=[pl.BlockSpec((2000, H), lambda i: (i, 0)),
                  pl.BlockSpec((1, H), lambda i: (0, 0)),
                  pl.BlockSpec((1, H), lambda i: (0, 0)),
                  pl.BlockSpec((1, H), lambda i: (0, 0)),
                  pl.BlockSpec((1, H), lambda i: (0, 0)),
                  pl.BlockSpec((H, 64), lambda i: (0, 0)),
                  pl.BlockSpec((1, 64), lambda i: (0, 0)),
                  pl.BlockSpec((64, 1), lambda i: (0, 0)),
                  pl.BlockSpec((1, 1), lambda i: (0, 0))],
        out_specs=[pl.BlockSpec((2000, 64), lambda i: (i, 0)),
                   pl.BlockSpec((2000, 1), lambda i: (i, 0))],
        out_shape=[jax.ShapeDtypeStruct((N, 64), jnp.float32),
                   jax.ShapeDtypeStruct((N, 1), jnp.float32)],
    )(h, mu, sig, g, be, wb, bb, wp, nrm)


BATCH_P = 2000


@functools.partial(
    pl.kernel, mesh=_mesh, compiler_params=_sc_params,
    out_type=[jax.ShapeDtypeStruct((TOT,), jnp.int32),
              jax.ShapeDtypeStruct((TOT,), jnp.int32)],
    scratch_types=[
        pltpu.VMEM((1024,), jnp.int32),
        pltpu.VMEM((64,), jnp.int32),
        pltpu.VMEM((BATCH_P,), jnp.int32),
        pltpu.VMEM((BATCH_P,), jnp.int32),
        pltpu.VMEM((BATCH_P,), jnp.int32),
        pltpu.SemaphoreType.DMA,
    ],
)
def _part_kernel(src_hbm, dst_hbm, rsrc_hbm, rdst_hbm,
                 padv, tails, srcv, dstv, spp, sem):
    c = lax.axis_index("c")
    s = lax.axis_index("s")
    p = s * 2 + c
    iota = _iota16()

    def _fill_src(g, _):
        padv[pl.ds(g * 16, 16)] = ((g * 16 + iota) * 977) & 65535
        return 0
    lax.fori_loop(0, 64, _fill_src, 0)
    for b in range(NBKT):
        pltpu.sync_copy(padv, rsrc_hbm.at[pl.ds((b * 32 + p.astype(jnp.int32)) * CAP, CAP)])

    def _fill_dst(g, b):
        padv[pl.ds(g * 16, 16)] = _i16(0) + (b + 1) * WBKT
        return b
    for b in range(NBKT):
        lax.fori_loop(0, 64, _fill_dst, b)
        pltpu.sync_copy(padv, rdst_hbm.at[pl.ds((b * 32 + p.astype(jnp.int32)) * CAP, CAP)])

    def _zt(g, _):
        tails[pl.ds(g * 16, 16)] = _i16(0)
        return 0
    lax.fori_loop(0, 4, _zt, 0)

    ebase = p * EPP

    def _batch(i, _):
        pltpu.sync_copy(src_hbm.at[pl.ds(ebase + i * BATCH_P, BATCH_P)], srcv)
        pltpu.sync_copy(dst_hbm.at[pl.ds(ebase + i * BATCH_P, BATCH_P)], dstv)

        def _edge(j, _):
            d = plsc.load_gather(dstv, [_i16(0) + j])
            b = lax.shift_right_logical((lax.shift_right_logical(d, 5)) * 5350, 18)
            tg = plsc.load_gather(tails, [b])
            pos = b * (32 * CAP) + (p * CAP) + tg
            plsc.store_scatter(spp, [_i16(0) + j], pos, mask=iota == 0)
            plsc.store_scatter(tails, [b], tg + 1, mask=iota == 0)
            return 0
        lax.fori_loop(0, BATCH_P, _edge, 0)
        pltpu.async_copy(srcv, rsrc_hbm.at[spp], sem).wait()
        pltpu.async_copy(dstv, rdst_hbm.at[spp], sem).wait()
        return 0
    lax.fori_loop(0, EPP // BATCH_P, _batch, 0)


BATCH_C = 512


@functools.partial(
    pl.kernel, mesh=_mesh, compiler_params=_sc_params,
    out_type=jax.ShapeDtypeStruct((NPAD, 16), jnp.float32),
    scratch_types=[
        pltpu.VMEM((WBKT + 1, 16), jnp.float32),
        pltpu.VMEM((WBKT + 1, 16), jnp.float32),
        pltpu.VMEM((WBKT + 1, 16), jnp.float32),
        pltpu.VMEM((BATCH_C,), jnp.int32),
        pltpu.VMEM((BATCH_C,), jnp.int32),
        pltpu.VMEM((BATCH_C, 16), jnp.float32),
        pltpu.SemaphoreType.DMA,
    ],
)
def _layer_kernel(msg_hbm, rsrc_hbm, rdst_hbm, zeros_hbm, num_hbm,
                  mmax, den, num, srcv, dlv, rows, sem):
    c = lax.axis_index("c")
    s = lax.axis_index("s")
    t = s * 2 + c
    iota = _iota16()
    nb = TOT // NBKT

    for q in range(2):
        b = t * 2 + q
        rbase = b * nb
        pltpu.sync_copy(zeros_hbm, mmax)
        pltpu.sync_copy(zeros_hbm, den)
        pltpu.sync_copy(zeros_hbm, num)

        def _load(i):
            pltpu.sync_copy(rsrc_hbm.at[pl.ds(rbase + i * BATCH_C, BATCH_C)], srcv)
            pltpu.sync_copy(rdst_hbm.at[pl.ds(rbase + i * BATCH_C, BATCH_C)], dlv)

            def _loc(g, _):
                dlv[pl.ds(g * 16, 16)] = dlv[pl.ds(g * 16, 16)] - b * WBKT
                return 0
            lax.fori_loop(0, BATCH_C // 16, _loc, 0)
            pltpu.async_copy(msg_hbm.at[srcv], rows, sem).wait()

        def _b0(i, _):
            _load(i)

            def _e0(j, _):
                locv = plsc.load_gather(dlv, [_i16(0) + j])
                r = rows[j, :]
                m = plsc.load_gather(mmax, [locv, iota])
                plsc.store_scatter(mmax, [locv, iota], jnp.maximum(m, r))
                return 0
            lax.fori_loop(0, BATCH_C, _e0, 0)
            return 0
        lax.fori_loop(0, nb // BATCH_C, _b0, 0)

        def _b1(i, _):
            _load(i)

            def _e1(j, _):
                locv = plsc.load_gather(dlv, [_i16(0) + j])
                r = rows[j, :]
                mm = plsc.load_gather(mmax, [locv, iota])
                ex = jnp.exp(r - mm)
                dd = plsc.load_gather(den, [locv, iota])
                plsc.store_scatter(den, [locv, iota], dd + ex)
                return 0
            lax.fori_loop(0, BATCH_C, _e1, 0)
            return 0
        lax.fori_loop(0, nb // BATCH_C, _b1, 0)

        def _b2(i, _):
            _load(i)

            def _e2(j, _):
                locv = plsc.load_gather(dlv, [_i16(0) + j])
                r = rows[j, :]
                mm = plsc.load_gather(mmax, [locv, iota])
                ex = jnp.exp(r - mm)
                dd = plsc.load_gather(den, [locv, iota])
                alpha = ex / jnp.maximum(dd, 1e-16)
                nn = plsc.load_gather(num, [locv, iota])
                plsc.store_scatter(num, [locv, iota], nn + r * alpha)
                return 0
            lax.fori_loop(0, BATCH_C, _e2, 0)
            return 0
        lax.fori_loop(0, nb // BATCH_C, _b2, 0)

        pltpu.sync_copy(num.at[pl.ds(0, WBKT)], num_hbm.at[pl.ds(b * WBKT, WBKT)])


NCAND = 768


@functools.partial(
    pl.kernel, mesh=_mesh, compiler_params=_sc_params,
    out_type=[jax.ShapeDtypeStruct((512, 64), jnp.float32),
              jax.ShapeDtypeStruct((NCAND,), jnp.int32),
              jax.ShapeDtypeStruct((NCAND,), jnp.int32)],
    scratch_types=[
        pltpu.VMEM((PT,), jnp.float32),
        pltpu.VMEM((PT,), jnp.int32),
        pltpu.VMEM((16,), jnp.int32),
        pltpu.VMEM((16, 16), jnp.int32),
        pltpu.VMEM_SHARED((16, 16), jnp.int32),
        pltpu.VMEM((128,), jnp.int32),
        pltpu.VMEM((128,), jnp.int32),
        pltpu.VMEM((128,), jnp.int32),
        pltpu.VMEM((512,), jnp.int32),
        pltpu.VMEM((512,), jnp.int32),
        pltpu.VMEM((32,), jnp.int32),
        pltpu.VMEM((32,), jnp.int32),
        pltpu.VMEM((32,), jnp.float32),
        pltpu.VMEM((32, 64), jnp.float32),
        pltpu.SemaphoreType.DMA,
    ],
)
def _topk_kernel(score_hbm, x3_hbm, out_hbm, ck_hbm, ci_hbm,
                 sv, kv, cst, crd, shc, stp, stk, sti, ckv, civ,
                 myci, mypos, myval, rows, sem):
    c = lax.axis_index("c")
    s = lax.axis_index("s")
    iota = _iota16()
    NG = PT // 16

    def _body():
        base = s * PT
        pltpu.sync_copy(score_hbm.at[pl.ds(base, PT)], sv)

        def _mkkey(g, _):
            x = sv[pl.ds(g * 16, 16)]
            x = jnp.where(x == 0.0, 0.0, x)
            bi = plsc.bitcast(x, jnp.int32)
            kv[pl.ds(g * 16, 16)] = jnp.where(bi < 0, bi ^ 0x7FFFFFFF, bi)
            return 0
        lax.fori_loop(0, NG, _mkkey, 0)

        def _count_gt(mid):
            def _cg(g, acc):
                k = kv[pl.ds(g * 16, 16)]
                return acc + plsc.all_reduce_population_count(k > mid)
            return lax.fori_loop(0, NG, _cg, _i16(0))

        def _exchange(vec):
            cst[...] = vec
            pltpu.sync_copy(cst, shc.at[s])
            plsc.subcore_barrier()
            pltpu.sync_copy(shc, crd)
            plsc.subcore_barrier()

        lo = _i16(-2147483647) - 1
        hi = _i16(2147483647)
        for _ in range(32):
            mid = lo + lax.shift_right_logical(hi - lo, 1)
            cnt = _count_gt(mid)
            _exchange(cnt)
            tot = _i16(0)
            for r in range(16):
                tot = tot + crd[r, :]
            ge = tot >= 512
            lo = jnp.where(ge, mid + 1, lo)
            hi = jnp.where(ge, hi, mid)
        vstar = lo

        def _c2(g, acc):
            k = kv[pl.ds(g * 16, 16)]
            gtc = plsc.all_reduce_population_count(k > vstar)
            eqc = plsc.all_reduce_population_count(k == vstar)
            return (acc[0] + gtc, acc[1] + eqc)
        gt_eq = lax.fori_loop(0, NG, _c2, (_i16(0), _i16(0)))
        mix = jnp.where(iota < 8, gt_eq[0], gt_eq[1])
        _exchange(mix)
        gt_base = _i16(0)
        eq_base = _i16(0)
        c_gt = _i16(0)
        sv16 = _i16(0) + s
        for r in range(16):
            row = crd[r, :]
            g_r = plsc.load_gather(crd, [_i16(r), _i16(0)])
            e_r = plsc.load_gather(crd, [_i16(r), _i16(8)])
            before = _i16(r) < sv16
            gt_base = gt_base + jnp.where(before, g_r, 0)
            eq_base = eq_base + jnp.where(before, e_r, 0)
            c_gt = c_gt + g_r
        quota = _i16(512) - c_gt

        trash = _i16(512) + sv16 * 16 + iota
        gt_run = _i16(0)
        eq_run = _i16(0)

        def _c3(g, carry):
            gt_run, eq_run = carry
            for u in range(8):
                gg = g * 8 + u
                k = kv[pl.ds(gg * 16, 16)]
                mgt = k > vstar
                meq = k == vstar
                rgt = plsc.cumsum(mgt.astype(jnp.int32))
                req = plsc.cumsum(meq.astype(jnp.int32))
                eqg = eq_base + eq_run + req - 1
                keep_eq = meq & (eqg < quota)
                pos = jnp.where(mgt, gt_base + gt_run + rgt - 1,
                                jnp.where(keep_eq, c_gt + eqg, trash))
                o = u * 16
                stp[pl.ds(o, 16)] = pos
                stk[pl.ds(o, 16)] = k
                sti[pl.ds(o, 16)] = _i16(base + gg * 16) + iota
                gt_run = gt_run + plsc.all_reduce_population_count(mgt)
                eq_run = eq_run + plsc.all_reduce_population_count(meq)
            pltpu.async_copy(stk, ck_hbm.at[stp], sem).wait()
            pltpu.async_copy(sti, ci_hbm.at[stp], sem).wait()
            return (gt_run, eq_run)
        carry = lax.fori_loop(0, 48, _c3, (gt_run, eq_run))
        gt_run, eq_run = carry
        for u in range(7):
            gg = 384 + u
            k = kv[pl.ds(gg * 16, 16)]
            mgt = k > vstar
            meq = k == vstar
            rgt = plsc.cumsum(mgt.astype(jnp.int32))
            req = plsc.cumsum(meq.astype(jnp.int32))
            eqg = eq_base + eq_run + req - 1
            keep_eq = meq & (eqg < quota)
            pos = jnp.where(mgt, gt_base + gt_run + rgt - 1,
                            jnp.where(keep_eq, c_gt + eqg, trash))
            o = u * 16
            stp[pl.ds(o, 16)] = pos
            stk[pl.ds(o, 16)] = k
            sti[pl.ds(o, 16)] = _i16(base + gg * 16) + iota
            gt_run = gt_run + plsc.all_reduce_population_count(mgt)
            eq_run = eq_run + plsc.all_reduce_population_count(meq)
        pltpu.async_copy(stk.at[pl.ds(0, 112)], ck_hbm.at[stp.at[pl.ds(0, 112)]], sem).wait()
        pltpu.async_copy(sti.at[pl.ds(0, 112)], ci_hbm.at[stp.at[pl.ds(0, 112)]], sem).wait()
        plsc.subcore_barrier()

        pltpu.sync_copy(ck_hbm.at[pl.ds(0, 512)], ckv)
        pltpu.sync_copy(ci_hbm.at[pl.ds(0, 512)], civ)
        pltpu.sync_copy(ci_hbm.at[pl.ds(0, 512)], civ)
        for ci_ in range(32):
            slot = sv16 * 32 + ci_
            kc = plsc.load_gather(ckv, [slot])
            ic = plsc.load_gather(civ, [slot])
            cnt = _i16(0)

            def _rank(bb, cnt):
                kb = ckv[pl.ds(bb * 16, 16)]
                ib = civ[pl.ds(bb * 16, 16)]
                before = (kb > kc) | ((kb == kc) & (ib < ic))
                return cnt + plsc.all_reduce_population_count(before)
            cnt = lax.fori_loop(0, 32, _rank, cnt)
            plsc.store_scatter(mypos, [_i16(ci_)], cnt, mask=iota == 0)
            plsc.store_scatter(myci, [_i16(ci_)], ic, mask=iota == 0)
            kneg = kc < 0
            bi = jnp.where(kneg, kc ^ 0x7FFFFFFF, kc)
            val = plsc.bitcast(bi, jnp.float32)
            plsc.store_scatter(myval, [_i16(ci_)], val, mask=iota == 0)

        pltpu.async_copy(x3_hbm.at[myci], rows, sem).wait()
        for ci_ in range(32):
            val = plsc.load_gather(myval, [_i16(ci_)])
            for qq in range(4):
                rows[ci_, pl.ds(qq * 16, 16)] = rows[ci_, pl.ds(qq * 16, 16)] * val
        pltpu.async_copy(rows, out_hbm.at[mypos], sem).wait()

    _body()


def kernel(pos, edge_index, W1a, b1a, g1, be1, W1b, b1b, W2a, b2a, g2, be2,
           W2b, b2b, w_pool):
    src = edge_index[0]
    dst = edge_index[1]

    x16, msg1 = _t1(pos)

    rsrc, rdst = _part_kernel(src, dst)

    zeros = jnp.zeros((WBKT + 1, 16), jnp.float32)
    num1 = _layer_kernel(msg1, rsrc, rdst, zeros)[:N]

    W1a_pad = jnp.concatenate(
        [W1a, jnp.zeros((13, W1a.shape[1]), W1a.dtype)], axis=0)
    out1, h1 = _t2(num1, x16, W1a_pad, b1a[None, :])
    mu1 = h1.mean(axis=0)
    var1 = h1.var(axis=0)
    sig1 = jnp.sqrt(var1 + 1e-5)
    x2, msg2 = _t3_l1(h1, mu1[None, :], sig1[None, :], g1[None, :],
                      be1[None, :], W1b, b1b[None, :])

    num2 = _layer_kernel(msg2, rsrc, rdst, zeros)[:N]
    out2, h2 = _t2(num2, x2, W2a, b2a[None, :])
    mu2 = h2.mean(axis=0)
    var2 = h2.var(axis=0)
    sig2 = jnp.sqrt(var2 + 1e-5)
    nrm = jnp.linalg.norm(w_pool)
    x3, score = _t3_l2(h2, mu2[None, :], sig2[None, :], g2[None, :],
                       be2[None, :], W2b, b2b[None, :], w_pool[:, None],
                       nrm[None, None])

    vals, perm = jax.lax.top_k(score[:, 0], 512)
    return x3[perm] * vals[:, None]
